# baseline (speedup 1.0000x reference)
import functools

import jax
import jax.numpy as jnp
from jax.experimental import pallas as pl
from jax.experimental.pallas import tpu as pltpu

_BT = 1024
_NBUF = 4


def _router_body(x_hbm, w_ref, b_ref, o_ref, *scratch, n_chunks):
    bufs = scratch[:_NBUF]
    sems = scratch[_NBUF]

    def start_copy(c):
        pltpu.make_async_copy(
            x_hbm.at[pl.ds(c * _BT, _BT), :],
            bufs[c % _NBUF],
            sems.at[c % _NBUF],
        ).start()

    for c in range(min(_NBUF - 1, n_chunks)):
        start_copy(c)

    for c in range(n_chunks):
        slot = c % _NBUF
        pltpu.make_async_copy(
            x_hbm.at[pl.ds(c * _BT, _BT), :],
            bufs[slot],
            sems.at[slot],
        ).wait()
        if c + _NBUF - 1 < n_chunks:
            start_copy(c + _NBUF - 1)
        logits = jax.lax.dot_general(
            w_ref[...], bufs[slot][...],
            dimension_numbers=(((1,), (1,)), ((), ())),
            preferred_element_type=jnp.float32,
        ) + b_ref[...][:, None]
        m = jnp.max(logits, axis=0, keepdims=True)
        e = jnp.exp(logits - m)
        o_ref[:, pl.ds(c * _BT, _BT)] = e / jnp.sum(e, axis=0, keepdims=True)


@jax.jit
def kernel(x, W, b):
    n_tokens, embed_dim = x.shape
    n_experts = W.shape[0]
    n_chunks = n_tokens // _BT
    out_t = pl.pallas_call(
        functools.partial(_router_body, n_chunks=n_chunks),
        in_specs=[
            pl.BlockSpec(memory_space=pltpu.MemorySpace.HBM),
            pl.BlockSpec(memory_space=pltpu.MemorySpace.VMEM),
            pl.BlockSpec(memory_space=pltpu.MemorySpace.VMEM),
        ],
        out_specs=pl.BlockSpec(memory_space=pltpu.MemorySpace.VMEM),
        out_shape=jax.ShapeDtypeStruct((n_experts, n_tokens), jnp.float32),
        scratch_shapes=(
            [pltpu.VMEM((_BT, embed_dim), jnp.float32) for _ in range(_NBUF)]
            + [pltpu.SemaphoreType.DMA((_NBUF,))]
        ),
    )(x, W, b)
    return out_t.T


# speedup vs baseline: 1.0575x; 1.0575x over previous
import jax
import jax.numpy as jnp
from jax.experimental import pallas as pl
from jax.experimental.pallas import tpu as pltpu

_BT = 1024


def _router_body(x_ref, w_ref, b_ref, o_ref):
    logits = jax.lax.dot_general(
        w_ref[...], x_ref[...],
        dimension_numbers=(((1,), (1,)), ((), ())),
        preferred_element_type=jnp.float32,
    ) + b_ref[...][:, None]
    m = jnp.max(logits, axis=0, keepdims=True)
    e = jnp.exp(logits - m)
    o_ref[...] = e / jnp.sum(e, axis=0, keepdims=True)


@jax.jit
def kernel(x, W, b):
    n_tokens, embed_dim = x.shape
    n_experts = W.shape[0]
    grid = (n_tokens // _BT,)
    out_t = pl.pallas_call(
        _router_body,
        grid=grid,
        in_specs=[
            pl.BlockSpec((_BT, embed_dim), lambda i: (i, 0)),
            pl.BlockSpec((n_experts, embed_dim), lambda i: (0, 0)),
            pl.BlockSpec((n_experts,), lambda i: (0,)),
        ],
        out_specs=pl.BlockSpec((n_experts, _BT), lambda i: (0, i)),
        out_shape=jax.ShapeDtypeStruct((n_experts, n_tokens), jnp.float32),
    )(x, W, b)
    return out_t.T


# speedup vs baseline: 1.0585x; 1.0010x over previous
import jax
import jax.numpy as jnp
from jax.experimental import pallas as pl

_BT = 1024


def _router_body(x_ref, w_ref, b_ref, o_ref):
    logits = jax.lax.dot_general(
        w_ref[...], x_ref[...],
        dimension_numbers=(((1,), (1,)), ((), ())),
        preferred_element_type=jnp.float32,
    ) + b_ref[...][:, None]
    m = jnp.max(logits, axis=0, keepdims=True)
    e = jnp.exp(logits - m)
    o_ref[...] = e / jnp.sum(e, axis=0, keepdims=True)


@jax.jit
def kernel(x, W, b):
    n_tokens, embed_dim = x.shape
    n_experts = W.shape[0]
    grid = (n_tokens // _BT,)
    out_t = pl.pallas_call(
        _router_body,
        grid=grid,
        in_specs=[
            pl.BlockSpec((_BT, embed_dim), lambda i: (i, 0)),
            pl.BlockSpec((n_experts, embed_dim), lambda i: (0, 0)),
            pl.BlockSpec((n_experts,), lambda i: (0,)),
        ],
        out_specs=pl.BlockSpec((n_experts, _BT), lambda i: (0, i)),
        out_shape=jax.ShapeDtypeStruct((n_experts, n_tokens), jnp.float32),
    )(x, W, b)
    return out_t.T
